# Initial kernel scaffold; baseline (speedup 1.0000x reference)
import jax
import jax.numpy as jnp
from jax.experimental import pallas as pl


def kernel(h, edge_index, edge_attr, W1, b1, W2, b2, W3, b3, W4, b4):
    raise NotImplementedError("write your pallas kernel here")


# speedup vs baseline: 2.5661x; 2.5661x over previous
import functools

import jax
import jax.numpy as jnp
from jax import lax
from jax.experimental import pallas as pl
from jax.experimental.pallas import tpu as pltpu
from jax.experimental.pallas import tpu_sc as plsc

N = 10000
D = 128
DE = 16
NC, NS = 2, 16
NW = NC * NS
CH = 128
NACC = 10240
NBLK = 1000
EBLK = 2048

_SC_MESH = plsc.VectorSubcoreMesh(
    core_axis_name="c", subcore_axis_name="s", num_cores=NC, num_subcores=NS)


def _node_proj_body(h_ref, w1a_ref, w1b_ref, p_ref, q_ref):
  hb = h_ref[...]
  p_ref[...] = jnp.dot(hb, w1a_ref[...], preferred_element_type=jnp.float32)
  q_ref[...] = jnp.dot(hb, w1b_ref[...], preferred_element_type=jnp.float32)


def _node_proj(h, w1a, w1b):
  return pl.pallas_call(
      _node_proj_body,
      grid=(N // NBLK,),
      in_specs=[
          pl.BlockSpec((NBLK, D), lambda i: (i, 0)),
          pl.BlockSpec((D, D), lambda i: (0, 0)),
          pl.BlockSpec((D, D), lambda i: (0, 0)),
      ],
      out_specs=(
          pl.BlockSpec((NBLK, D), lambda i: (i, 0)),
          pl.BlockSpec((NBLK, D), lambda i: (i, 0)),
      ),
      out_shape=(
          jax.ShapeDtypeStruct((N, D), jnp.float32),
          jax.ShapeDtypeStruct((N, D), jnp.float32),
      ),
  )(h, w1a, w1b)


def _gather_body(p_hbm, q_hbm, ridx_hbm, cidx_hbm, x_hbm,
                 ridx_v, cidx_v, bufp, bufq, semp, semq):
  nchunk = ridx_v.shape[0]
  ew = nchunk * CH
  wid = lax.axis_index("s") * NC + lax.axis_index("c")
  base = wid * ew
  pltpu.sync_copy(ridx_hbm.at[wid], ridx_v)
  pltpu.sync_copy(cidx_hbm.at[wid], cidx_v)

  def chunk(j, carry):
    cp = pltpu.async_copy(p_hbm.at[ridx_v.at[j]], bufp, semp)
    cq = pltpu.async_copy(q_hbm.at[cidx_v.at[j]], bufq, semq)
    cp.wait()
    cq.wait()

    def row(i, c2):
      for k in range(D // 16):
        s = pl.ds(k * 16, 16)
        bufp[i, s] = bufp[i, s] + bufq[i, s]
      return c2

    lax.fori_loop(0, CH, row, 0)
    pltpu.sync_copy(bufp, x_hbm.at[pl.ds(base + j * CH, CH)])
    return carry

  lax.fori_loop(0, nchunk, chunk, 0)


def _gather_add(p, q, ridx3, cidx3, e_pad):
  nchunk = ridx3.shape[1]
  return pl.kernel(
      _gather_body,
      out_type=jax.ShapeDtypeStruct((e_pad, D), jnp.float32),
      mesh=_SC_MESH,
      scratch_types=[
          pltpu.VMEM((nchunk, CH), jnp.int32),
          pltpu.VMEM((nchunk, CH), jnp.int32),
          pltpu.VMEM((CH, D), jnp.float32),
          pltpu.VMEM((CH, D), jnp.float32),
          pltpu.SemaphoreType.DMA,
          pltpu.SemaphoreType.DMA,
      ],
  )(p, q, ridx3, cidx3)


def _edge_mlp_body(x_ref, ea_ref, w1c_ref, b1_ref, w2_ref, b2_ref, m_ref):
  t = (x_ref[...]
       + jnp.dot(ea_ref[...], w1c_ref[...], preferred_element_type=jnp.float32)
       + b1_ref[...])
  t = jnp.maximum(t, 0.0)
  t = jnp.dot(t, w2_ref[...], preferred_element_type=jnp.float32) + b2_ref[...]
  m_ref[...] = jnp.maximum(t, 0.0)


def _edge_mlp(x, ea, w1c, b1r, w2, b2r, e_pad):
  return pl.pallas_call(
      _edge_mlp_body,
      grid=(e_pad // EBLK,),
      in_specs=[
          pl.BlockSpec((EBLK, D), lambda i: (i, 0)),
          pl.BlockSpec((EBLK, DE), lambda i: (i, 0)),
          pl.BlockSpec((DE, D), lambda i: (0, 0)),
          pl.BlockSpec((1, D), lambda i: (0, 0)),
          pl.BlockSpec((D, D), lambda i: (0, 0)),
          pl.BlockSpec((1, D), lambda i: (0, 0)),
      ],
      out_specs=pl.BlockSpec((EBLK, D), lambda i: (i, 0)),
      out_shape=jax.ShapeDtypeStruct((e_pad, D), jnp.float32),
  )(x, ea, w1c, b1r, w2, b2r)


def _scatter_body(m_hbm, sidx_hbm, part_hbm, sidx_v, bufm, zbuf, acc, sem):
  nchunk = sidx_v.shape[0]
  ew = nchunk * CH
  cid = lax.axis_index("c")
  sid = lax.axis_index("s")
  wid = sid * NC + cid
  base = wid * ew
  rows_per_tile = NACC // NS

  def zrow(i, c2):
    for k in range(D // 16):
      zbuf[i, pl.ds(k * 16, 16)] = jnp.zeros((16,), jnp.float32)
    return c2

  lax.fori_loop(0, CH, zrow, 0)
  for t in range(rows_per_tile // CH):
    pltpu.sync_copy(zbuf, acc.at[pl.ds(sid * rows_per_tile + t * CH, CH)])
  plsc.subcore_barrier()

  pltpu.sync_copy(sidx_hbm.at[wid], sidx_v)

  def chunk(j, carry):
    pltpu.sync_copy(m_hbm.at[pl.ds(base + j * CH, CH)], bufm)
    pltpu.sync_copy(bufm, acc.at[sidx_v.at[j]], add=True)
    return carry

  lax.fori_loop(0, nchunk, chunk, 0)
  plsc.subcore_barrier()
  pltpu.sync_copy(acc.at[pl.ds(sid * rows_per_tile, rows_per_tile)],
                  part_hbm.at[cid, pl.ds(sid * rows_per_tile, rows_per_tile)])


def _segment_sum(m, sidx3):
  nchunk = sidx3.shape[1]
  return pl.kernel(
      _scatter_body,
      out_type=jax.ShapeDtypeStruct((NC, NACC, D), jnp.float32),
      mesh=_SC_MESH,
      scratch_types=[
          pltpu.VMEM((nchunk, CH), jnp.int32),
          pltpu.VMEM((CH, D), jnp.float32),
          pltpu.VMEM((CH, D), jnp.float32),
          pltpu.VMEM_SHARED((NACC, D), jnp.float32),
          pltpu.SemaphoreType.DMA,
      ],
  )(m, sidx3)


def _node_update_body(h_ref, m0_ref, m1_ref, w3_ref, b3_ref, w4_ref, b4_ref,
                      o_ref):
  hb = h_ref[...]
  m = m0_ref[0] + m1_ref[0]
  agg = jnp.concatenate([hb, m], axis=1)
  t = jnp.maximum(
      jnp.dot(agg, w3_ref[...], preferred_element_type=jnp.float32)
      + b3_ref[...], 0.0)
  o_ref[...] = (hb
                + jnp.dot(t, w4_ref[...], preferred_element_type=jnp.float32)
                + b4_ref[...])


def _node_update(h, partials, w3, b3r, w4, b4r):
  return pl.pallas_call(
      _node_update_body,
      grid=(N // NBLK,),
      in_specs=[
          pl.BlockSpec((NBLK, D), lambda i: (i, 0)),
          pl.BlockSpec((1, NBLK, D), lambda i: (0, i, 0)),
          pl.BlockSpec((1, NBLK, D), lambda i: (1, i, 0)),
          pl.BlockSpec((2 * D, D), lambda i: (0, 0)),
          pl.BlockSpec((1, D), lambda i: (0, 0)),
          pl.BlockSpec((D, D), lambda i: (0, 0)),
          pl.BlockSpec((1, D), lambda i: (0, 0)),
      ],
      out_specs=pl.BlockSpec((NBLK, D), lambda i: (i, 0)),
      out_shape=jax.ShapeDtypeStruct((N, D), jnp.float32),
  )(h, partials, partials, w3, b3r, w4, b4r)


def kernel(h, edge_index, edge_attr, W1, b1, W2, b2, W3, b3, W4, b4):
  e = edge_attr.shape[0]
  nchunk = -(-e // (NW * CH))
  e_pad = NW * nchunk * CH
  pad = e_pad - e

  r = edge_index[0].astype(jnp.int32)
  c = edge_index[1].astype(jnp.int32)
  ridx3 = jnp.pad(r, (0, pad)).reshape(NW, nchunk, CH)
  cidx3 = jnp.pad(c, (0, pad)).reshape(NW, nchunk, CH)
  sidx3 = jnp.pad(r, (0, pad), constant_values=N).reshape(NW, nchunk, CH)
  ea_pad = jnp.pad(edge_attr, ((0, pad), (0, 0)))

  w1a = W1[:D]
  w1b = W1[D:2 * D]
  w1c = W1[2 * D:]
  b1r = b1.reshape(1, D)
  b2r = b2.reshape(1, D)
  b3r = b3.reshape(1, D)
  b4r = b4.reshape(1, D)

  p, q = _node_proj(h, w1a, w1b)
  x = _gather_add(p, q, ridx3, cidx3, e_pad)
  m = _edge_mlp(x, ea_pad, w1c, b1r, W2, b2r, e_pad)
  partials = _segment_sum(m, sidx3)
  return _node_update(h, partials, W3, b3r, W4, b4r)
